# Initial kernel scaffold; baseline (speedup 1.0000x reference)
import jax
import jax.numpy as jnp
from jax.experimental import pallas as pl


def kernel(x, params):
    raise NotImplementedError("write your pallas kernel here")


# speedup vs baseline: 9.4719x; 9.4719x over previous
import functools

import jax
import jax.numpy as jnp
from jax import lax
from jax.experimental import pallas as pl
from jax.experimental.pallas import tpu as pltpu
from jax.experimental.pallas import tpu_sc as plsc

_TILE = 256
_KTILE = 128


def _round_up(a, m):
    return (a + m - 1) // m * m


def _knn(h_p, sq_p, n, k):
    NP, d = h_p.shape
    tile = _KTILE
    grid = NP // tile

    def body(h_blk, hT_ref, sqr, sqc, out_ref):
        pid = pl.program_id(0)
        p = jnp.dot(h_blk[...], hT_ref[...])
        s = (sqc[...] - 2.0 * p) + sqr[...]
        col = lax.broadcasted_iota(jnp.int32, (tile, NP), 1)
        row = lax.broadcasted_iota(jnp.int32, (tile, NP), 0) + pid * tile
        s = jnp.where((col == row) | (col >= n), jnp.inf, s)
        cols_out = []
        for kk in range(k):
            idx = jnp.argmin(s, axis=1).astype(jnp.int32)
            cols_out.append(idx[:, None])
            if kk + 1 < k:
                s = jnp.where(col == idx[:, None], jnp.inf, s)
        out_ref[...] = jnp.concatenate(cols_out, axis=1)

    return pl.pallas_call(
        body,
        grid=(grid,),
        in_specs=[
            pl.BlockSpec((tile, d), lambda i: (i, 0)),
            pl.BlockSpec((d, NP), lambda i: (0, 0)),
            pl.BlockSpec((1, NP), lambda i: (0, 0)),
            pl.BlockSpec((tile, 1), lambda i: (i, 0)),
        ],
        out_specs=pl.BlockSpec((tile, k), lambda i: (i, 0)),
        out_shape=jax.ShapeDtypeStruct((NP, k), jnp.int32),
    )(h_p, h_p.T, sq_p[None, :], sq_p[:, None])


def _sc_gather(table, idx):
    B = idx.shape[0]
    V, D = table.shape
    info = plsc.get_sparse_core_info()
    nw = info.num_cores * info.num_subcores
    b_per_w = B // nw
    mesh = plsc.VectorSubcoreMesh(core_axis_name="c", subcore_axis_name="s")

    @functools.partial(
        pl.kernel,
        mesh=mesh,
        compiler_params=pltpu.CompilerParams(use_tc_tiling_on_sc=False),
        out_type=jax.ShapeDtypeStruct((B, D), jnp.float32),
        scratch_types=[
            pltpu.VMEM((b_per_w,), jnp.int32),
            pltpu.VMEM((b_per_w, D), jnp.float32),
            pltpu.SemaphoreType.DMA,
        ],
    )
    def gk(table_hbm, idx_hbm, out_hbm, idx_v, rows_v, sem):
        wid = lax.axis_index("s") * info.num_cores + lax.axis_index("c")
        base = wid * b_per_w
        pltpu.sync_copy(idx_hbm.at[pl.ds(base, b_per_w)], idx_v)
        pltpu.async_copy(table_hbm.at[idx_v], rows_v, sem).wait()
        pltpu.sync_copy(rows_v, out_hbm.at[pl.ds(base, b_per_w)])

    return gk(table, idx)


def _encoder(x_p, xj_cat, enc, k, d, dp):
    NP = x_p.shape[0]
    (w1, b1), (w2, b2) = enc
    tile = _TILE

    def body(xi_ref, xj_ref, w1r, b1r, w2r, b2r, out_ref):
        xi = xi_ref[...]
        acc = None
        for kk in range(k):
            xjk = xj_ref[:, kk * dp:kk * dp + d]
            inp = jnp.concatenate([xi, xjk - xi], axis=1)
            hid = jnp.maximum(jnp.dot(inp, w1r[...]) + b1r[...], 0.0)
            msg = jnp.dot(hid, w2r[...]) + b2r[...]
            acc = msg if acc is None else acc + msg
        out_ref[...] = jnp.maximum(acc, 0.0)

    return pl.pallas_call(
        body,
        grid=(NP // tile,),
        in_specs=[
            pl.BlockSpec((tile, d), lambda i: (i, 0)),
            pl.BlockSpec((tile, k * dp), lambda i: (i, 0)),
            pl.BlockSpec(w1.shape, lambda i: (0, 0)),
            pl.BlockSpec((1, b1.shape[0]), lambda i: (0, 0)),
            pl.BlockSpec(w2.shape, lambda i: (0, 0)),
            pl.BlockSpec((1, b2.shape[0]), lambda i: (0, 0)),
        ],
        out_specs=pl.BlockSpec((tile, w2.shape[1]), lambda i: (i, 0)),
        out_shape=jax.ShapeDtypeStruct((NP, w2.shape[1]), jnp.float32),
    )(x_p, xj_cat, w1, b1[None, :], w2, b2[None, :])


def _edge_enc(h, hj, enc, k):
    NP, hd = h.shape
    (w1, b1), (w2, b2) = enc
    ed = w2.shape[1]
    tile = _TILE

    def body(h_ref, hj_ref, w1r, b1r, w2r, b2r, out_ref):
        hi = h_ref[...]
        outs = []
        for kk in range(k):
            hjk = hj_ref[:, kk * 16:kk * 16 + hd]
            inp = jnp.concatenate([hjk, hi], axis=1)
            hid = jnp.maximum(jnp.dot(inp, w1r[...]) + b1r[...], 0.0)
            outs.append(jnp.maximum(jnp.dot(hid, w2r[...]) + b2r[...], 0.0))
        out_ref[...] = jnp.concatenate(outs, axis=1)

    return pl.pallas_call(
        body,
        grid=(NP // tile,),
        in_specs=[
            pl.BlockSpec((tile, hd), lambda i: (i, 0)),
            pl.BlockSpec((tile, k * 16), lambda i: (i, 0)),
            pl.BlockSpec(w1.shape, lambda i: (0, 0)),
            pl.BlockSpec((1, b1.shape[0]), lambda i: (0, 0)),
            pl.BlockSpec(w2.shape, lambda i: (0, 0)),
            pl.BlockSpec((1, b2.shape[0]), lambda i: (0, 0)),
        ],
        out_specs=pl.BlockSpec((tile, k * ed), lambda i: (i, 0)),
        out_shape=jax.ShapeDtypeStruct((NP, k * ed), jnp.float32),
    )(h, hj, w1, b1[None, :], w2, b2[None, :])


def _in_layer(h, hj, e, lp, k, alpha):
    NP, hd = h.shape
    (rw1, rb1), (rw2, rb2), (rw3, rb3) = lp["rel"]
    (ow1, ob1), (ow2, ob2), (ow3, ob3) = lp["obj"]
    ed = rw3.shape[1]
    tile = _TILE

    def body(h_ref, hj_ref, e_ref,
             rw1r, rb1r, rw2r, rb2r, rw3r, rb3r,
             ow1r, ob1r, ow2r, ob2r, ow3r, ob3r,
             h_out, e_out):
        hi = h_ref[...]
        outs = []
        agg = None
        for kk in range(k):
            hjk = hj_ref[:, kk * 16:kk * 16 + hd]
            ek = e_ref[:, kk * ed:(kk + 1) * ed]
            inp = jnp.concatenate([hi, hjk, ek], axis=1)
            t = jnp.maximum(jnp.dot(inp, rw1r[...]) + rb1r[...], 0.0)
            t = jnp.maximum(jnp.dot(t, rw2r[...]) + rb2r[...], 0.0)
            et = jnp.dot(t, rw3r[...]) + rb3r[...]
            outs.append(et)
            agg = et if agg is None else agg + et
        inp2 = jnp.concatenate([hi, agg], axis=1)
        t = jnp.maximum(jnp.dot(inp2, ow1r[...]) + ob1r[...], 0.0)
        t = jnp.maximum(jnp.dot(t, ow2r[...]) + ob2r[...], 0.0)
        dh = jnp.dot(t, ow3r[...]) + ob3r[...]
        h_out[...] = alpha * hi + (1.0 - alpha) * dh
        e_out[...] = jnp.concatenate(outs, axis=1)

    full = lambda w: pl.BlockSpec(w.shape, lambda i: (0, 0))
    bias = lambda b: pl.BlockSpec((1, b.shape[0]), lambda i: (0, 0))
    return pl.pallas_call(
        body,
        grid=(NP // tile,),
        in_specs=[
            pl.BlockSpec((tile, hd), lambda i: (i, 0)),
            pl.BlockSpec((tile, k * 16), lambda i: (i, 0)),
            pl.BlockSpec((tile, k * ed), lambda i: (i, 0)),
            full(rw1), bias(rb1), full(rw2), bias(rb2), full(rw3), bias(rb3),
            full(ow1), bias(ob1), full(ow2), bias(ob2), full(ow3), bias(ob3),
        ],
        out_specs=[
            pl.BlockSpec((tile, hd), lambda i: (i, 0)),
            pl.BlockSpec((tile, k * ed), lambda i: (i, 0)),
        ],
        out_shape=[
            jax.ShapeDtypeStruct((NP, hd), jnp.float32),
            jax.ShapeDtypeStruct((NP, k * ed), jnp.float32),
        ],
    )(h, hj, e,
      rw1, rb1[None, :], rw2, rb2[None, :], rw3, rb3[None, :],
      ow1, ob1[None, :], ow2, ob2[None, :], ow3, ob3[None, :])


def _final(h, bparams, xparams):
    NP, hd = h.shape
    (bw1, bb1), (bw2, bb2), (bw3, bb3) = bparams
    (xw1, xb1), (xw2, xb2), (xw3, xb3) = xparams
    tile = _TILE

    def body(h_ref,
             bw1r, bb1r, bw2r, bb2r, bw3r, bb3r,
             xw1r, xb1r, xw2r, xb2r, xw3r, xb3r,
             hout_ref, beta_ref):
        hi = h_ref[...]
        t = jnp.maximum(jnp.dot(hi, bw1r[...]) + bb1r[...], 0.0)
        t = jnp.maximum(jnp.dot(t, bw2r[...]) + bb2r[...], 0.0)
        blog = jnp.dot(t, bw3r[...]) + bb3r[...]
        beta_ref[...] = jax.nn.sigmoid(blog) + 1e-11
        t = jnp.maximum(jnp.dot(hi, xw1r[...]) + xb1r[...], 0.0)
        t = jnp.maximum(jnp.dot(t, xw2r[...]) + xb2r[...], 0.0)
        hout_ref[...] = jnp.dot(t, xw3r[...]) + xb3r[...]

    full = lambda w: pl.BlockSpec(w.shape, lambda i: (0, 0))
    bias = lambda b: pl.BlockSpec((1, b.shape[0]), lambda i: (0, 0))
    return pl.pallas_call(
        body,
        grid=(NP // tile,),
        in_specs=[
            pl.BlockSpec((tile, hd), lambda i: (i, 0)),
            full(bw1), bias(bb1), full(bw2), bias(bb2), full(bw3), bias(bb3),
            full(xw1), bias(xb1), full(xw2), bias(xb2), full(xw3), bias(xb3),
        ],
        out_specs=[
            pl.BlockSpec((tile, xw3.shape[1]), lambda i: (i, 0)),
            pl.BlockSpec((tile, 1), lambda i: (i, 0)),
        ],
        out_shape=[
            jax.ShapeDtypeStruct((NP, xw3.shape[1]), jnp.float32),
            jax.ShapeDtypeStruct((NP, 1), jnp.float32),
        ],
    )(h,
      bw1, bb1[None, :], bw2, bb2[None, :], bw3, bb3[None, :],
      xw1, xb1[None, :], xw2, xb2[None, :], xw3, xb3[None, :])


def _block(bp, h, k, alpha):
    n, d = h.shape
    NP = _round_up(n, 512)
    sq = jnp.sum(h * h, axis=1)
    h_p = jnp.pad(h, ((0, NP - n), (0, 0)))
    sq_p = jnp.pad(sq, (0, NP - n))

    nbr = _knn(h_p, sq_p, n, k)
    idx = nbr.reshape(-1)

    dp = d if d % 16 == 0 else _round_up(d, 16)
    tab = h_p if dp == d else jnp.pad(h_p, ((0, 0), (0, dp - d)))
    xj = _sc_gather(tab, idx).reshape(NP, k * dp)
    h1 = _encoder(h_p, xj, bp["node_encoder"], k, d, dp)

    hd = h1.shape[1]
    hdp = _round_up(hd, 16)
    hj = _sc_gather(jnp.pad(h1, ((0, 0), (0, hdp - hd))), idx).reshape(NP, k * hdp)
    e = _edge_enc(h1, hj, bp["edge_encoder"], k)

    hcur = h1
    nlayers = len(bp["layers"])
    for li, lp in enumerate(bp["layers"]):
        hcur, e = _in_layer(hcur, hj, e, lp, k, alpha)
        if li + 1 < nlayers:
            hj = _sc_gather(jnp.pad(hcur, ((0, 0), (0, hdp - hd))), idx).reshape(NP, k * hdp)
    return hcur[:n]


def kernel(x, params):
    alpha = 0.5
    nb = len(params["blocks"])
    ks = [nb - 1] + [nb - 1 - i for i in range(nb - 1)]
    h = x
    for bp, k in zip(params["blocks"], ks):
        h = _block(bp, h, k, alpha)
    n = h.shape[0]
    NP = _round_up(n, 512)
    h_p = jnp.pad(h, ((0, NP - n), (0, 0)))
    h_out, beta = _final(h_p, params["B"], params["X"])
    return (h_out[:n], beta[:n])


# speedup vs baseline: 11.2673x; 1.1895x over previous
import functools

import jax
import jax.numpy as jnp
from jax import lax
from jax.experimental import pallas as pl
from jax.experimental.pallas import tpu as pltpu
from jax.experimental.pallas import tpu_sc as plsc

_TILE = 512
_KTILE = 256


def _round_up(a, m):
    return (a + m - 1) // m * m


def _knn(h_p, sq_p, n, k):
    NP, d = h_p.shape
    tile = _KTILE
    grid = NP // tile

    def body(h_blk, hT_ref, sqr, sqc, out_ref):
        pid = pl.program_id(0)
        p = jnp.dot(h_blk[...], hT_ref[...])
        s = (sqc[...] - 2.0 * p) + sqr[...]
        col = lax.broadcasted_iota(jnp.int32, (tile, NP), 1)
        row = lax.broadcasted_iota(jnp.int32, (tile, NP), 0) + pid * tile
        s = jnp.where(col == row, jnp.inf, s)
        cols_out = []
        for kk in range(k):
            idx = jnp.argmin(s, axis=1).astype(jnp.int32)
            cols_out.append(idx[:, None])
            if kk + 1 < k:
                s = jnp.where(col == idx[:, None], jnp.inf, s)
        out_ref[...] = jnp.concatenate(cols_out, axis=1)

    return pl.pallas_call(
        body,
        grid=(grid,),
        in_specs=[
            pl.BlockSpec((tile, d), lambda i: (i, 0)),
            pl.BlockSpec((d, NP), lambda i: (0, 0)),
            pl.BlockSpec((1, NP), lambda i: (0, 0)),
            pl.BlockSpec((tile, 1), lambda i: (i, 0)),
        ],
        out_specs=pl.BlockSpec((tile, k), lambda i: (i, 0)),
        out_shape=jax.ShapeDtypeStruct((NP, k), jnp.int32),
    )(h_p, h_p.T, sq_p[None, :], sq_p[:, None])


def _sc_gather(table, idx):
    B = idx.shape[0]
    V, D = table.shape
    info = plsc.get_sparse_core_info()
    nw = info.num_cores * info.num_subcores
    b_per_w = B // nw
    mesh = plsc.VectorSubcoreMesh(core_axis_name="c", subcore_axis_name="s")

    @functools.partial(
        pl.kernel,
        mesh=mesh,
        compiler_params=pltpu.CompilerParams(use_tc_tiling_on_sc=False),
        out_type=jax.ShapeDtypeStruct((B, D), jnp.float32),
        scratch_types=[
            pltpu.VMEM((b_per_w,), jnp.int32),
            pltpu.VMEM((b_per_w, D), jnp.float32),
            pltpu.SemaphoreType.DMA,
        ],
    )
    def gk(table_hbm, idx_hbm, out_hbm, idx_v, rows_v, sem):
        wid = lax.axis_index("s") * info.num_cores + lax.axis_index("c")
        base = wid * b_per_w
        pltpu.sync_copy(idx_hbm.at[pl.ds(base, b_per_w)], idx_v)
        pltpu.async_copy(table_hbm.at[idx_v], rows_v, sem).wait()
        pltpu.sync_copy(rows_v, out_hbm.at[pl.ds(base, b_per_w)])

    return gk(table, idx)


def _encoder(x_p, xj_cat, enc, k, d, dp):
    NP = x_p.shape[0]
    (w1, b1), (w2, b2) = enc
    tile = _TILE

    def body(xi_ref, xj_ref, w1r, b1r, w2r, b2r, out_ref):
        xi = xi_ref[...]
        acc = None
        for kk in range(k):
            xjk = xj_ref[:, kk * dp:kk * dp + d]
            inp = jnp.concatenate([xi, xjk - xi], axis=1)
            hid = jnp.maximum(jnp.dot(inp, w1r[...]) + b1r[...], 0.0)
            msg = jnp.dot(hid, w2r[...]) + b2r[...]
            acc = msg if acc is None else acc + msg
        out_ref[...] = jnp.maximum(acc, 0.0)

    return pl.pallas_call(
        body,
        grid=(NP // tile,),
        in_specs=[
            pl.BlockSpec((tile, d), lambda i: (i, 0)),
            pl.BlockSpec((tile, k * dp), lambda i: (i, 0)),
            pl.BlockSpec(w1.shape, lambda i: (0, 0)),
            pl.BlockSpec((1, b1.shape[0]), lambda i: (0, 0)),
            pl.BlockSpec(w2.shape, lambda i: (0, 0)),
            pl.BlockSpec((1, b2.shape[0]), lambda i: (0, 0)),
        ],
        out_specs=pl.BlockSpec((tile, w2.shape[1]), lambda i: (i, 0)),
        out_shape=jax.ShapeDtypeStruct((NP, w2.shape[1]), jnp.float32),
    )(x_p, xj_cat, w1, b1[None, :], w2, b2[None, :])


def _edge_enc(h, hj, enc, k):
    NP, hd = h.shape
    (w1, b1), (w2, b2) = enc
    ed = w2.shape[1]
    tile = _TILE

    def body(h_ref, hj_ref, w1r, b1r, w2r, b2r, out_ref):
        hi = h_ref[...]
        outs = []
        for kk in range(k):
            hjk = hj_ref[:, kk * 16:kk * 16 + hd]
            inp = jnp.concatenate([hjk, hi], axis=1)
            hid = jnp.maximum(jnp.dot(inp, w1r[...]) + b1r[...], 0.0)
            outs.append(jnp.maximum(jnp.dot(hid, w2r[...]) + b2r[...], 0.0))
        out_ref[...] = jnp.concatenate(outs, axis=1)

    return pl.pallas_call(
        body,
        grid=(NP // tile,),
        in_specs=[
            pl.BlockSpec((tile, hd), lambda i: (i, 0)),
            pl.BlockSpec((tile, k * 16), lambda i: (i, 0)),
            pl.BlockSpec(w1.shape, lambda i: (0, 0)),
            pl.BlockSpec((1, b1.shape[0]), lambda i: (0, 0)),
            pl.BlockSpec(w2.shape, lambda i: (0, 0)),
            pl.BlockSpec((1, b2.shape[0]), lambda i: (0, 0)),
        ],
        out_specs=pl.BlockSpec((tile, k * ed), lambda i: (i, 0)),
        out_shape=jax.ShapeDtypeStruct((NP, k * ed), jnp.float32),
    )(h, hj, w1, b1[None, :], w2, b2[None, :])


def _in_layer(h, hj, e, lp, k, alpha):
    NP, hd = h.shape
    (rw1, rb1), (rw2, rb2), (rw3, rb3) = lp["rel"]
    (ow1, ob1), (ow2, ob2), (ow3, ob3) = lp["obj"]
    ed = rw3.shape[1]
    tile = _TILE

    def body(h_ref, hj_ref, e_ref,
             rw1r, rb1r, rw2r, rb2r, rw3r, rb3r,
             ow1r, ob1r, ow2r, ob2r, ow3r, ob3r,
             h_out, e_out):
        hi = h_ref[...]
        outs = []
        agg = None
        for kk in range(k):
            hjk = hj_ref[:, kk * 16:kk * 16 + hd]
            ek = e_ref[:, kk * ed:(kk + 1) * ed]
            inp = jnp.concatenate([hi, hjk, ek], axis=1)
            t = jnp.maximum(jnp.dot(inp, rw1r[...]) + rb1r[...], 0.0)
            t = jnp.maximum(jnp.dot(t, rw2r[...]) + rb2r[...], 0.0)
            et = jnp.dot(t, rw3r[...]) + rb3r[...]
            outs.append(et)
            agg = et if agg is None else agg + et
        inp2 = jnp.concatenate([hi, agg], axis=1)
        t = jnp.maximum(jnp.dot(inp2, ow1r[...]) + ob1r[...], 0.0)
        t = jnp.maximum(jnp.dot(t, ow2r[...]) + ob2r[...], 0.0)
        dh = jnp.dot(t, ow3r[...]) + ob3r[...]
        h_out[...] = alpha * hi + (1.0 - alpha) * dh
        e_out[...] = jnp.concatenate(outs, axis=1)

    full = lambda w: pl.BlockSpec(w.shape, lambda i: (0, 0))
    bias = lambda b: pl.BlockSpec((1, b.shape[0]), lambda i: (0, 0))
    return pl.pallas_call(
        body,
        grid=(NP // tile,),
        in_specs=[
            pl.BlockSpec((tile, hd), lambda i: (i, 0)),
            pl.BlockSpec((tile, k * 16), lambda i: (i, 0)),
            pl.BlockSpec((tile, k * ed), lambda i: (i, 0)),
            full(rw1), bias(rb1), full(rw2), bias(rb2), full(rw3), bias(rb3),
            full(ow1), bias(ob1), full(ow2), bias(ob2), full(ow3), bias(ob3),
        ],
        out_specs=[
            pl.BlockSpec((tile, hd), lambda i: (i, 0)),
            pl.BlockSpec((tile, k * ed), lambda i: (i, 0)),
        ],
        out_shape=[
            jax.ShapeDtypeStruct((NP, hd), jnp.float32),
            jax.ShapeDtypeStruct((NP, k * ed), jnp.float32),
        ],
    )(h, hj, e,
      rw1, rb1[None, :], rw2, rb2[None, :], rw3, rb3[None, :],
      ow1, ob1[None, :], ow2, ob2[None, :], ow3, ob3[None, :])


def _final(h, bparams, xparams):
    NP, hd = h.shape
    (bw1, bb1), (bw2, bb2), (bw3, bb3) = bparams
    (xw1, xb1), (xw2, xb2), (xw3, xb3) = xparams
    tile = _TILE

    def body(h_ref,
             bw1r, bb1r, bw2r, bb2r, bw3r, bb3r,
             xw1r, xb1r, xw2r, xb2r, xw3r, xb3r,
             hout_ref, beta_ref):
        hi = h_ref[...]
        t = jnp.maximum(jnp.dot(hi, bw1r[...]) + bb1r[...], 0.0)
        t = jnp.maximum(jnp.dot(t, bw2r[...]) + bb2r[...], 0.0)
        blog = jnp.dot(t, bw3r[...]) + bb3r[...]
        beta_ref[...] = jax.nn.sigmoid(blog) + 1e-11
        t = jnp.maximum(jnp.dot(hi, xw1r[...]) + xb1r[...], 0.0)
        t = jnp.maximum(jnp.dot(t, xw2r[...]) + xb2r[...], 0.0)
        hout_ref[...] = jnp.dot(t, xw3r[...]) + xb3r[...]

    full = lambda w: pl.BlockSpec(w.shape, lambda i: (0, 0))
    bias = lambda b: pl.BlockSpec((1, b.shape[0]), lambda i: (0, 0))
    return pl.pallas_call(
        body,
        grid=(NP // tile,),
        in_specs=[
            pl.BlockSpec((tile, hd), lambda i: (i, 0)),
            full(bw1), bias(bb1), full(bw2), bias(bb2), full(bw3), bias(bb3),
            full(xw1), bias(xb1), full(xw2), bias(xb2), full(xw3), bias(xb3),
        ],
        out_specs=[
            pl.BlockSpec((tile, xw3.shape[1]), lambda i: (i, 0)),
            pl.BlockSpec((tile, 1), lambda i: (i, 0)),
        ],
        out_shape=[
            jax.ShapeDtypeStruct((NP, xw3.shape[1]), jnp.float32),
            jax.ShapeDtypeStruct((NP, 1), jnp.float32),
        ],
    )(h,
      bw1, bb1[None, :], bw2, bb2[None, :], bw3, bb3[None, :],
      xw1, xb1[None, :], xw2, xb2[None, :], xw3, xb3[None, :])


def _block(bp, h, k, alpha):
    n, d = h.shape
    NP = _round_up(n, 512)
    sq = jnp.sum(h * h, axis=1)
    h_p = jnp.pad(h, ((0, NP - n), (0, 0)))
    sq_p = jnp.pad(sq, (0, NP - n), constant_values=jnp.inf)

    nbr = _knn(h_p, sq_p, n, k)
    idx = nbr.reshape(-1)

    dp = d if d % 16 == 0 else _round_up(d, 16)
    tab = h_p if dp == d else jnp.pad(h_p, ((0, 0), (0, dp - d)))
    xj = _sc_gather(tab, idx).reshape(NP, k * dp)
    h1 = _encoder(h_p, xj, bp["node_encoder"], k, d, dp)

    hd = h1.shape[1]
    hdp = _round_up(hd, 16)
    hj = _sc_gather(jnp.pad(h1, ((0, 0), (0, hdp - hd))), idx).reshape(NP, k * hdp)
    e = _edge_enc(h1, hj, bp["edge_encoder"], k)

    hcur = h1
    nlayers = len(bp["layers"])
    for li, lp in enumerate(bp["layers"]):
        hcur, e = _in_layer(hcur, hj, e, lp, k, alpha)
        if li + 1 < nlayers:
            hj = _sc_gather(jnp.pad(hcur, ((0, 0), (0, hdp - hd))), idx).reshape(NP, k * hdp)
    return hcur[:n]


def kernel(x, params):
    alpha = 0.5
    nb = len(params["blocks"])
    ks = [nb - 1] + [nb - 1 - i for i in range(nb - 1)]
    h = x
    for bp, k in zip(params["blocks"], ks):
        h = _block(bp, h, k, alpha)
    n = h.shape[0]
    NP = _round_up(n, 512)
    h_p = jnp.pad(h, ((0, NP - n), (0, 0)))
    h_out, beta = _final(h_p, params["B"], params["X"])
    return (h_out[:n], beta[:n])


# speedup vs baseline: 11.6977x; 1.0382x over previous
import functools

import jax
import jax.numpy as jnp
from jax import lax
from jax.experimental import pallas as pl
from jax.experimental.pallas import tpu as pltpu
from jax.experimental.pallas import tpu_sc as plsc

_TILE = 512
_KTILE = 256
_HP = 16


def _round_up(a, m):
    return (a + m - 1) // m * m


def _knn(h_p, sq_p, k):
    NP, d = h_p.shape
    tile = _KTILE
    grid = NP // tile

    def body(h_blk, hT_ref, sqr, sqc, out_ref):
        pid = pl.program_id(0)
        p = jnp.dot(h_blk[...], hT_ref[...])
        s = (sqc[...] - 2.0 * p) + sqr[...]
        col = lax.broadcasted_iota(jnp.int32, (tile, NP), 1)
        row = lax.broadcasted_iota(jnp.int32, (tile, NP), 0) + pid * tile
        s = jnp.where(col == row, jnp.inf, s)
        cols_out = []
        for kk in range(k):
            idx = jnp.argmin(s, axis=1).astype(jnp.int32)
            cols_out.append(idx[:, None])
            if kk + 1 < k:
                s = jnp.where(col == idx[:, None], jnp.inf, s)
        out_ref[...] = jnp.concatenate(cols_out, axis=1)

    return pl.pallas_call(
        body,
        grid=(grid,),
        in_specs=[
            pl.BlockSpec((tile, d), lambda i: (i, 0)),
            pl.BlockSpec((d, NP), lambda i: (0, 0)),
            pl.BlockSpec((1, NP), lambda i: (0, 0)),
            pl.BlockSpec((tile, 1), lambda i: (i, 0)),
        ],
        out_specs=pl.BlockSpec((tile, k), lambda i: (i, 0)),
        out_shape=jax.ShapeDtypeStruct((NP, k), jnp.int32),
    )(h_p, h_p.T, sq_p[None, :], sq_p[:, None])


def _sc_gather(table, idx):
    B = idx.shape[0]
    V, D = table.shape
    info = plsc.get_sparse_core_info()
    nw = info.num_cores * info.num_subcores
    b_per_w = B // nw
    mesh = plsc.VectorSubcoreMesh(core_axis_name="c", subcore_axis_name="s")

    @functools.partial(
        pl.kernel,
        mesh=mesh,
        compiler_params=pltpu.CompilerParams(use_tc_tiling_on_sc=False),
        out_type=jax.ShapeDtypeStruct((B, D), jnp.float32),
        scratch_types=[
            pltpu.VMEM((b_per_w,), jnp.int32),
            pltpu.VMEM((b_per_w, D), jnp.float32),
            pltpu.SemaphoreType.DMA,
        ],
    )
    def gk(table_hbm, idx_hbm, out_hbm, idx_v, rows_v, sem):
        wid = lax.axis_index("s") * info.num_cores + lax.axis_index("c")
        base = wid * b_per_w
        pltpu.sync_copy(idx_hbm.at[pl.ds(base, b_per_w)], idx_v)
        pltpu.async_copy(table_hbm.at[idx_v], rows_v, sem).wait()
        pltpu.sync_copy(rows_v, out_hbm.at[pl.ds(base, b_per_w)])

    return gk(table, idx)


def _encoder(x_p, xj_cat, enc, k, dp):
    NP = x_p.shape[0]
    (w1, b1), (w2, b2) = enc
    d = w1.shape[0] // 2
    hd = w2.shape[1]
    tile = _TILE

    def body(xi_ref, xj_ref, w1r, b1r, w2r, b2r, out_ref):
        xi = xi_ref[:, :d]
        acc = None
        for kk in range(k):
            xjk = xj_ref[:, kk * dp:kk * dp + d]
            inp = jnp.concatenate([xi, xjk - xi], axis=1)
            hid = jnp.maximum(jnp.dot(inp, w1r[...]) + b1r[...], 0.0)
            msg = jnp.dot(hid, w2r[...]) + b2r[...]
            acc = msg if acc is None else acc + msg
        h = jnp.maximum(acc, 0.0)
        out_ref[...] = jnp.concatenate(
            [h, jnp.zeros((tile, _HP - hd), jnp.float32)], axis=1)

    return pl.pallas_call(
        body,
        grid=(NP // tile,),
        in_specs=[
            pl.BlockSpec((tile, x_p.shape[1]), lambda i: (i, 0)),
            pl.BlockSpec((tile, k * dp), lambda i: (i, 0)),
            pl.BlockSpec(w1.shape, lambda i: (0, 0)),
            pl.BlockSpec((1, b1.shape[0]), lambda i: (0, 0)),
            pl.BlockSpec(w2.shape, lambda i: (0, 0)),
            pl.BlockSpec((1, b2.shape[0]), lambda i: (0, 0)),
        ],
        out_specs=pl.BlockSpec((tile, _HP), lambda i: (i, 0)),
        out_shape=jax.ShapeDtypeStruct((NP, _HP), jnp.float32),
    )(x_p, xj_cat, w1, b1[None, :], w2, b2[None, :])


def _in_layer(h, hj, e, lp, k, alpha, eenc=None):
    NP = h.shape[0]
    (rw1, rb1), (rw2, rb2), (rw3, rb3) = lp["rel"]
    (ow1, ob1), (ow2, ob2), (ow3, ob3) = lp["obj"]
    ed = rw3.shape[1]
    hd = ow3.shape[1]
    tile = _TILE

    def compute(hi16, hjs, eks, wr):
        hi = hi16[:, :hd]
        outs = []
        agg = None
        for kk in range(k):
            hjk = hjs[kk]
            if eenc is None:
                ek = eks[kk]
            else:
                ei = jnp.concatenate([hjk, hi], axis=1)
                t = jnp.maximum(jnp.dot(ei, wr["ew1"][...]) + wr["eb1"][...], 0.0)
                ek = jnp.maximum(jnp.dot(t, wr["ew2"][...]) + wr["eb2"][...], 0.0)
            inp = jnp.concatenate([hi, hjk, ek], axis=1)
            t = jnp.maximum(jnp.dot(inp, wr["rw1"][...]) + wr["rb1"][...], 0.0)
            t = jnp.maximum(jnp.dot(t, wr["rw2"][...]) + wr["rb2"][...], 0.0)
            et = jnp.dot(t, wr["rw3"][...]) + wr["rb3"][...]
            outs.append(et)
            agg = et if agg is None else agg + et
        inp2 = jnp.concatenate([hi, agg], axis=1)
        t = jnp.maximum(jnp.dot(inp2, wr["ow1"][...]) + wr["ob1"][...], 0.0)
        t = jnp.maximum(jnp.dot(t, wr["ow2"][...]) + wr["ob2"][...], 0.0)
        dh = jnp.dot(t, wr["ow3"][...]) + wr["ob3"][...]
        hn = alpha * hi + (1.0 - alpha) * dh
        hn16 = jnp.concatenate(
            [hn, jnp.zeros((tile, _HP - hd), jnp.float32)], axis=1)
        return hn16, jnp.concatenate(outs, axis=1)

    names = ["rw1", "rb1", "rw2", "rb2", "rw3", "rb3",
             "ow1", "ob1", "ow2", "ob2", "ow3", "ob3"]
    weights = [rw1, rb1[None, :], rw2, rb2[None, :], rw3, rb3[None, :],
               ow1, ob1[None, :], ow2, ob2[None, :], ow3, ob3[None, :]]
    if eenc is not None:
        (ew1, eb1), (ew2, eb2) = eenc
        names += ["ew1", "eb1", "ew2", "eb2"]
        weights += [ew1, eb1[None, :], ew2, eb2[None, :]]

    def body(*refs):
        if eenc is None:
            h_ref, hj_ref, e_ref = refs[:3]
            wrefs = refs[3:-2]
        else:
            h_ref, hj_ref = refs[:2]
            e_ref = None
            wrefs = refs[2:-2]
        h_out, e_out = refs[-2], refs[-1]
        wr = dict(zip(names, wrefs))
        hjs = [hj_ref[:, kk * _HP:kk * _HP + hd] for kk in range(k)]
        eks = (None if e_ref is None else
               [e_ref[:, kk * ed:(kk + 1) * ed] for kk in range(k)])
        hn16, e_new = compute(h_ref[...], hjs, eks, wr)
        h_out[...] = hn16
        e_out[...] = e_new

    row = lambda w: pl.BlockSpec((tile, w), lambda i: (i, 0))
    full = lambda w: pl.BlockSpec(w.shape, lambda i: (0, 0))
    in_specs = [row(_HP), row(k * _HP)]
    operands = [h, hj]
    if eenc is None:
        in_specs.append(row(k * ed))
        operands.append(e)
    in_specs += [full(w) for w in weights]
    operands += weights
    return pl.pallas_call(
        body,
        grid=(NP // tile,),
        in_specs=in_specs,
        out_specs=[row(_HP), row(k * ed)],
        out_shape=[
            jax.ShapeDtypeStruct((NP, _HP), jnp.float32),
            jax.ShapeDtypeStruct((NP, k * ed), jnp.float32),
        ],
    )(*operands)


def _final(h, bparams, xparams):
    NP = h.shape[0]
    (bw1, bb1), (bw2, bb2), (bw3, bb3) = bparams
    (xw1, xb1), (xw2, xb2), (xw3, xb3) = xparams
    hd = bw1.shape[0]
    tile = _TILE

    def body(h_ref,
             bw1r, bb1r, bw2r, bb2r, bw3r, bb3r,
             xw1r, xb1r, xw2r, xb2r, xw3r, xb3r,
             hout_ref, beta_ref):
        hi = h_ref[:, :hd]
        t = jnp.maximum(jnp.dot(hi, bw1r[...]) + bb1r[...], 0.0)
        t = jnp.maximum(jnp.dot(t, bw2r[...]) + bb2r[...], 0.0)
        blog = jnp.dot(t, bw3r[...]) + bb3r[...]
        beta_ref[...] = jax.nn.sigmoid(blog) + 1e-11
        t = jnp.maximum(jnp.dot(hi, xw1r[...]) + xb1r[...], 0.0)
        t = jnp.maximum(jnp.dot(t, xw2r[...]) + xb2r[...], 0.0)
        hout_ref[...] = jnp.dot(t, xw3r[...]) + xb3r[...]

    full = lambda w: pl.BlockSpec(w.shape, lambda i: (0, 0))
    bias = lambda b: pl.BlockSpec((1, b.shape[0]), lambda i: (0, 0))
    return pl.pallas_call(
        body,
        grid=(NP // tile,),
        in_specs=[
            pl.BlockSpec((tile, _HP), lambda i: (i, 0)),
            full(bw1), bias(bb1), full(bw2), bias(bb2), full(bw3), bias(bb3),
            full(xw1), bias(xb1), full(xw2), bias(xb2), full(xw3), bias(xb3),
        ],
        out_specs=[
            pl.BlockSpec((tile, xw3.shape[1]), lambda i: (i, 0)),
            pl.BlockSpec((tile, 1), lambda i: (i, 0)),
        ],
        out_shape=[
            jax.ShapeDtypeStruct((NP, xw3.shape[1]), jnp.float32),
            jax.ShapeDtypeStruct((NP, 1), jnp.float32),
        ],
    )(h,
      bw1, bb1[None, :], bw2, bb2[None, :], bw3, bb3[None, :],
      xw1, xb1[None, :], xw2, xb2[None, :], xw3, xb3[None, :])


def _block(bp, h_p, n, k, alpha):
    NP, dp = h_p.shape
    d = bp["node_encoder"][0][0].shape[0] // 2
    sq = jnp.sum(h_p[:n] * h_p[:n], axis=1)
    sq_p = jnp.pad(sq, (0, NP - n), constant_values=jnp.inf)

    nbr = _knn(h_p, sq_p, k)
    idx = nbr.reshape(-1)

    xj = _sc_gather(h_p, idx).reshape(NP, k * dp)
    h1 = _encoder(h_p, xj, bp["node_encoder"], k, dp)

    hj = _sc_gather(h1, idx).reshape(NP, k * _HP)
    hcur, e = _in_layer(h1, hj, None, bp["layers"][0], k, alpha,
                        eenc=bp["edge_encoder"])
    for lp in bp["layers"][1:]:
        hj = _sc_gather(hcur, idx).reshape(NP, k * _HP)
        hcur, e = _in_layer(hcur, hj, e, lp, k, alpha)
    return hcur


def kernel(x, params):
    alpha = 0.5
    nb = len(params["blocks"])
    ks = [nb - 1] + [nb - 1 - i for i in range(nb - 1)]
    n = x.shape[0]
    NP = _round_up(n, 512)
    h_p = jnp.pad(x, ((0, NP - n), (0, 0)))
    for bp, k in zip(params["blocks"], ks):
        h_p = _block(bp, h_p, n, k, alpha)
    h_out, beta = _final(h_p, params["B"], params["X"])
    return (h_out[:n], beta[:n])


# speedup vs baseline: 12.0705x; 1.0319x over previous
import functools

import jax
import jax.numpy as jnp
from jax import lax
from jax.experimental import pallas as pl
from jax.experimental.pallas import tpu as pltpu
from jax.experimental.pallas import tpu_sc as plsc

_TILE = 512
_KTILE = 512
_HP = 16


def _round_up(a, m):
    return (a + m - 1) // m * m


def _knn(h_p, sq_p, k):
    NP, d = h_p.shape
    tile = _KTILE
    grid = NP // tile

    def body(h_blk, hT_ref, sqr, sqc, out_ref):
        pid = pl.program_id(0)
        p = jnp.dot(h_blk[...], hT_ref[...])
        s = (sqc[...] - 2.0 * p) + sqr[...]
        col = lax.broadcasted_iota(jnp.int32, (tile, NP), 1)
        row = lax.broadcasted_iota(jnp.int32, (tile, NP), 0) + pid * tile
        s = jnp.where(col == row, jnp.inf, s)
        cols_out = []
        for kk in range(k):
            idx = jnp.argmin(s, axis=1).astype(jnp.int32)
            cols_out.append(idx[:, None])
            if kk + 1 < k:
                s = jnp.where(col == idx[:, None], jnp.inf, s)
        out_ref[...] = jnp.concatenate(cols_out, axis=1)

    return pl.pallas_call(
        body,
        grid=(grid,),
        in_specs=[
            pl.BlockSpec((tile, d), lambda i: (i, 0)),
            pl.BlockSpec((d, NP), lambda i: (0, 0)),
            pl.BlockSpec((1, NP), lambda i: (0, 0)),
            pl.BlockSpec((tile, 1), lambda i: (i, 0)),
        ],
        out_specs=pl.BlockSpec((tile, k), lambda i: (i, 0)),
        out_shape=jax.ShapeDtypeStruct((NP, k), jnp.int32),
    )(h_p, h_p.T, sq_p[None, :], sq_p[:, None])


def _sc_gather(table, idx):
    B = idx.shape[0]
    V, D = table.shape
    info = plsc.get_sparse_core_info()
    nw = info.num_cores * info.num_subcores
    b_per_w = B // nw
    mesh = plsc.VectorSubcoreMesh(core_axis_name="c", subcore_axis_name="s")

    @functools.partial(
        pl.kernel,
        mesh=mesh,
        compiler_params=pltpu.CompilerParams(use_tc_tiling_on_sc=False),
        out_type=jax.ShapeDtypeStruct((B, D), jnp.float32),
        scratch_types=[
            pltpu.VMEM((b_per_w,), jnp.int32),
            pltpu.VMEM((b_per_w, D), jnp.float32),
            pltpu.SemaphoreType.DMA,
        ],
    )
    def gk(table_hbm, idx_hbm, out_hbm, idx_v, rows_v, sem):
        wid = lax.axis_index("s") * info.num_cores + lax.axis_index("c")
        base = wid * b_per_w
        pltpu.sync_copy(idx_hbm.at[pl.ds(base, b_per_w)], idx_v)
        pltpu.async_copy(table_hbm.at[idx_v], rows_v, sem).wait()
        pltpu.sync_copy(rows_v, out_hbm.at[pl.ds(base, b_per_w)])

    return gk(table, idx)


def _encoder(x_p, xj_cat, enc, k, dp):
    NP = x_p.shape[0]
    (w1, b1), (w2, b2) = enc
    d = w1.shape[0] // 2
    hd = w2.shape[1]
    tile = _TILE

    def body(xi_ref, xj_ref, w1r, b1r, w2r, b2r, out_ref):
        xi = xi_ref[:, :d]
        acc = None
        for kk in range(k):
            xjk = xj_ref[:, kk * dp:kk * dp + d]
            inp = jnp.concatenate([xi, xjk - xi], axis=1)
            hid = jnp.maximum(jnp.dot(inp, w1r[...]) + b1r[...], 0.0)
            msg = jnp.dot(hid, w2r[...]) + b2r[...]
            acc = msg if acc is None else acc + msg
        h = jnp.maximum(acc, 0.0)
        out_ref[...] = jnp.concatenate(
            [h, jnp.zeros((tile, _HP - hd), jnp.float32)], axis=1)

    return pl.pallas_call(
        body,
        grid=(NP // tile,),
        in_specs=[
            pl.BlockSpec((tile, x_p.shape[1]), lambda i: (i, 0)),
            pl.BlockSpec((tile, k * dp), lambda i: (i, 0)),
            pl.BlockSpec(w1.shape, lambda i: (0, 0)),
            pl.BlockSpec((1, b1.shape[0]), lambda i: (0, 0)),
            pl.BlockSpec(w2.shape, lambda i: (0, 0)),
            pl.BlockSpec((1, b2.shape[0]), lambda i: (0, 0)),
        ],
        out_specs=pl.BlockSpec((tile, _HP), lambda i: (i, 0)),
        out_shape=jax.ShapeDtypeStruct((NP, _HP), jnp.float32),
    )(x_p, xj_cat, w1, b1[None, :], w2, b2[None, :])


def _in_layer(h, hj, e, lp, k, alpha, eenc=None):
    NP = h.shape[0]
    (rw1, rb1), (rw2, rb2), (rw3, rb3) = lp["rel"]
    (ow1, ob1), (ow2, ob2), (ow3, ob3) = lp["obj"]
    ed = rw3.shape[1]
    hd = ow3.shape[1]
    tile = _TILE

    def compute(hi16, hjs, eks, wr):
        hi = hi16[:, :hd]
        outs = []
        agg = None
        for kk in range(k):
            hjk = hjs[kk]
            if eenc is None:
                ek = eks[kk]
            else:
                ei = jnp.concatenate([hjk, hi], axis=1)
                t = jnp.maximum(jnp.dot(ei, wr["ew1"][...]) + wr["eb1"][...], 0.0)
                ek = jnp.maximum(jnp.dot(t, wr["ew2"][...]) + wr["eb2"][...], 0.0)
            inp = jnp.concatenate([hi, hjk, ek], axis=1)
            t = jnp.maximum(jnp.dot(inp, wr["rw1"][...]) + wr["rb1"][...], 0.0)
            t = jnp.maximum(jnp.dot(t, wr["rw2"][...]) + wr["rb2"][...], 0.0)
            et = jnp.dot(t, wr["rw3"][...]) + wr["rb3"][...]
            outs.append(et)
            agg = et if agg is None else agg + et
        inp2 = jnp.concatenate([hi, agg], axis=1)
        t = jnp.maximum(jnp.dot(inp2, wr["ow1"][...]) + wr["ob1"][...], 0.0)
        t = jnp.maximum(jnp.dot(t, wr["ow2"][...]) + wr["ob2"][...], 0.0)
        dh = jnp.dot(t, wr["ow3"][...]) + wr["ob3"][...]
        hn = alpha * hi + (1.0 - alpha) * dh
        hn16 = jnp.concatenate(
            [hn, jnp.zeros((tile, _HP - hd), jnp.float32)], axis=1)
        return hn16, jnp.concatenate(outs, axis=1)

    names = ["rw1", "rb1", "rw2", "rb2", "rw3", "rb3",
             "ow1", "ob1", "ow2", "ob2", "ow3", "ob3"]
    weights = [rw1, rb1[None, :], rw2, rb2[None, :], rw3, rb3[None, :],
               ow1, ob1[None, :], ow2, ob2[None, :], ow3, ob3[None, :]]
    if eenc is not None:
        (ew1, eb1), (ew2, eb2) = eenc
        names += ["ew1", "eb1", "ew2", "eb2"]
        weights += [ew1, eb1[None, :], ew2, eb2[None, :]]

    def body(*refs):
        if eenc is None:
            h_ref, hj_ref, e_ref = refs[:3]
            wrefs = refs[3:-2]
        else:
            h_ref, hj_ref = refs[:2]
            e_ref = None
            wrefs = refs[2:-2]
        h_out, e_out = refs[-2], refs[-1]
        wr = dict(zip(names, wrefs))
        hjs = [hj_ref[:, kk * _HP:kk * _HP + hd] for kk in range(k)]
        eks = (None if e_ref is None else
               [e_ref[:, kk * ed:(kk + 1) * ed] for kk in range(k)])
        hn16, e_new = compute(h_ref[...], hjs, eks, wr)
        h_out[...] = hn16
        e_out[...] = e_new

    row = lambda w: pl.BlockSpec((tile, w), lambda i: (i, 0))
    full = lambda w: pl.BlockSpec(w.shape, lambda i: (0, 0))
    in_specs = [row(_HP), row(k * _HP)]
    operands = [h, hj]
    if eenc is None:
        in_specs.append(row(k * ed))
        operands.append(e)
    in_specs += [full(w) for w in weights]
    operands += weights
    return pl.pallas_call(
        body,
        grid=(NP // tile,),
        in_specs=in_specs,
        out_specs=[row(_HP), row(k * ed)],
        out_shape=[
            jax.ShapeDtypeStruct((NP, _HP), jnp.float32),
            jax.ShapeDtypeStruct((NP, k * ed), jnp.float32),
        ],
    )(*operands)


def _final(h, bparams, xparams):
    NP = h.shape[0]
    (bw1, bb1), (bw2, bb2), (bw3, bb3) = bparams
    (xw1, xb1), (xw2, xb2), (xw3, xb3) = xparams
    hd = bw1.shape[0]
    tile = _TILE

    def body(h_ref,
             bw1r, bb1r, bw2r, bb2r, bw3r, bb3r,
             xw1r, xb1r, xw2r, xb2r, xw3r, xb3r,
             hout_ref, beta_ref):
        hi = h_ref[:, :hd]
        t = jnp.maximum(jnp.dot(hi, bw1r[...]) + bb1r[...], 0.0)
        t = jnp.maximum(jnp.dot(t, bw2r[...]) + bb2r[...], 0.0)
        blog = jnp.dot(t, bw3r[...]) + bb3r[...]
        beta_ref[...] = jax.nn.sigmoid(blog) + 1e-11
        t = jnp.maximum(jnp.dot(hi, xw1r[...]) + xb1r[...], 0.0)
        t = jnp.maximum(jnp.dot(t, xw2r[...]) + xb2r[...], 0.0)
        hout_ref[...] = jnp.dot(t, xw3r[...]) + xb3r[...]

    full = lambda w: pl.BlockSpec(w.shape, lambda i: (0, 0))
    bias = lambda b: pl.BlockSpec((1, b.shape[0]), lambda i: (0, 0))
    return pl.pallas_call(
        body,
        grid=(NP // tile,),
        in_specs=[
            pl.BlockSpec((tile, _HP), lambda i: (i, 0)),
            full(bw1), bias(bb1), full(bw2), bias(bb2), full(bw3), bias(bb3),
            full(xw1), bias(xb1), full(xw2), bias(xb2), full(xw3), bias(xb3),
        ],
        out_specs=[
            pl.BlockSpec((tile, xw3.shape[1]), lambda i: (i, 0)),
            pl.BlockSpec((tile, 1), lambda i: (i, 0)),
        ],
        out_shape=[
            jax.ShapeDtypeStruct((NP, xw3.shape[1]), jnp.float32),
            jax.ShapeDtypeStruct((NP, 1), jnp.float32),
        ],
    )(h,
      bw1, bb1[None, :], bw2, bb2[None, :], bw3, bb3[None, :],
      xw1, xb1[None, :], xw2, xb2[None, :], xw3, xb3[None, :])


def _block(bp, h_p, n, k, alpha):
    NP, dp = h_p.shape
    d = bp["node_encoder"][0][0].shape[0] // 2
    hr = h_p[:n, :d]
    sq = jnp.sum(hr * hr, axis=1)
    sq_p = jnp.pad(sq, (0, NP - n), constant_values=jnp.inf)

    nbr = _knn(h_p, sq_p, k)
    idx = nbr.reshape(-1)

    xj = _sc_gather(h_p, idx).reshape(NP, k * dp)
    h1 = _encoder(h_p, xj, bp["node_encoder"], k, dp)

    hj = _sc_gather(h1, idx).reshape(NP, k * _HP)
    hcur, e = _in_layer(h1, hj, None, bp["layers"][0], k, alpha,
                        eenc=bp["edge_encoder"])
    for lp in bp["layers"][1:]:
        hj = _sc_gather(hcur, idx).reshape(NP, k * _HP)
        hcur, e = _in_layer(hcur, hj, e, lp, k, alpha)
    return hcur


def kernel(x, params):
    alpha = 0.5
    nb = len(params["blocks"])
    ks = [nb - 1] + [nb - 1 - i for i in range(nb - 1)]
    n = x.shape[0]
    NP = _round_up(n, 512)
    h_p = jnp.pad(x, ((0, NP - n), (0, 0)))
    for bp, k in zip(params["blocks"], ks):
        h_p = _block(bp, h_p, n, k, alpha)
    h_out, beta = _final(h_p, params["B"], params["X"])
    return (h_out[:n], beta[:n])


# speedup vs baseline: 12.1387x; 1.0056x over previous
import functools

import jax
import jax.numpy as jnp
from jax import lax
from jax.experimental import pallas as pl
from jax.experimental.pallas import tpu as pltpu
from jax.experimental.pallas import tpu_sc as plsc

_TILE = 512
_KTILE = 512
_HP = 16


def _round_up(a, m):
    return (a + m - 1) // m * m


def _knn(h_p, sq_p, k):
    NP, d = h_p.shape
    tile = _KTILE
    grid = NP // tile

    def body(h_blk, hT_ref, sqr, sqc, out_ref):
        pid = pl.program_id(0)
        p = jnp.dot(h_blk[...], hT_ref[...])
        s = (sqc[...] - 2.0 * p) + sqr[...]
        col = lax.broadcasted_iota(jnp.int32, (tile, NP), 1)
        row = lax.broadcasted_iota(jnp.int32, (tile, NP), 0) + pid * tile
        s = jnp.where(col == row, jnp.inf, s)
        cols_out = []
        for kk in range(k):
            idx = jnp.argmin(s, axis=1).astype(jnp.int32)
            cols_out.append(idx[:, None])
            if kk + 1 < k:
                s = jnp.where(col == idx[:, None], jnp.inf, s)
        out_ref[...] = jnp.concatenate(cols_out, axis=1)

    return pl.pallas_call(
        body,
        grid=(grid,),
        in_specs=[
            pl.BlockSpec((tile, d), lambda i: (i, 0)),
            pl.BlockSpec((d, NP), lambda i: (0, 0)),
            pl.BlockSpec((1, NP), lambda i: (0, 0)),
            pl.BlockSpec((tile, 1), lambda i: (i, 0)),
        ],
        out_specs=pl.BlockSpec((tile, k), lambda i: (i, 0)),
        out_shape=jax.ShapeDtypeStruct((NP, k), jnp.int32),
    )(h_p, h_p.T, sq_p[None, :], sq_p[:, None])


def _sc_gather(table, idx):
    B = idx.shape[0]
    V, D = table.shape
    info = plsc.get_sparse_core_info()
    nw = info.num_cores * info.num_subcores
    b_per_w = B // nw
    mesh = plsc.VectorSubcoreMesh(core_axis_name="c", subcore_axis_name="s")

    @functools.partial(
        pl.kernel,
        mesh=mesh,
        compiler_params=pltpu.CompilerParams(use_tc_tiling_on_sc=False),
        out_type=jax.ShapeDtypeStruct((B, D), jnp.float32),
        scratch_types=[
            pltpu.VMEM((b_per_w,), jnp.int32),
            pltpu.VMEM((b_per_w, D), jnp.float32),
            pltpu.SemaphoreType.DMA,
        ],
    )
    def gk(table_hbm, idx_hbm, out_hbm, idx_v, rows_v, sem):
        wid = lax.axis_index("s") * info.num_cores + lax.axis_index("c")
        base = wid * b_per_w
        pltpu.sync_copy(idx_hbm.at[pl.ds(base, b_per_w)], idx_v)
        pltpu.async_copy(table_hbm.at[idx_v], rows_v, sem).wait()
        pltpu.sync_copy(rows_v, out_hbm.at[pl.ds(base, b_per_w)])

    return gk(table, idx)


def _encoder(x_p, xj_cat, enc, k, dp):
    NP = x_p.shape[0]
    (w1, b1), (w2, b2) = enc
    d = w1.shape[0] // 2
    hd = w2.shape[1]
    tile = _TILE

    def body(xi_ref, xj_ref, w1r, b1r, w2r, b2r, out_ref):
        xi = xi_ref[:, :d]
        acc = None
        for kk in range(k):
            xjk = xj_ref[:, kk * dp:kk * dp + d]
            inp = jnp.concatenate([xi, xjk - xi], axis=1)
            hid = jnp.maximum(jnp.dot(inp, w1r[...]) + b1r[...], 0.0)
            msg = jnp.dot(hid, w2r[...]) + b2r[...]
            acc = msg if acc is None else acc + msg
        h = jnp.maximum(acc, 0.0)
        out_ref[...] = jnp.concatenate(
            [h, jnp.zeros((tile, _HP - hd), jnp.float32)], axis=1)

    return pl.pallas_call(
        body,
        grid=(NP // tile,),
        in_specs=[
            pl.BlockSpec((tile, x_p.shape[1]), lambda i: (i, 0)),
            pl.BlockSpec((tile, k * dp), lambda i: (i, 0)),
            pl.BlockSpec(w1.shape, lambda i: (0, 0)),
            pl.BlockSpec((1, b1.shape[0]), lambda i: (0, 0)),
            pl.BlockSpec(w2.shape, lambda i: (0, 0)),
            pl.BlockSpec((1, b2.shape[0]), lambda i: (0, 0)),
        ],
        out_specs=pl.BlockSpec((tile, _HP), lambda i: (i, 0)),
        out_shape=jax.ShapeDtypeStruct((NP, _HP), jnp.float32),
    )(x_p, xj_cat, w1, b1[None, :], w2, b2[None, :])


def _in_layer(h, hj, e, lp, k, alpha, eenc=None, want_e=True):
    NP = h.shape[0]
    (rw1, rb1), (rw2, rb2), (rw3, rb3) = lp["rel"]
    (ow1, ob1), (ow2, ob2), (ow3, ob3) = lp["obj"]
    ed = rw3.shape[1]
    hd = ow3.shape[1]
    tile = _TILE

    def compute(hi16, hjs, eks, wr):
        hi = hi16[:, :hd]
        outs = []
        agg = None
        for kk in range(k):
            hjk = hjs[kk]
            if eenc is None:
                ek = eks[kk]
            else:
                ei = jnp.concatenate([hjk, hi], axis=1)
                t = jnp.maximum(jnp.dot(ei, wr["ew1"][...]) + wr["eb1"][...], 0.0)
                ek = jnp.maximum(jnp.dot(t, wr["ew2"][...]) + wr["eb2"][...], 0.0)
            inp = jnp.concatenate([hi, hjk, ek], axis=1)
            t = jnp.maximum(jnp.dot(inp, wr["rw1"][...]) + wr["rb1"][...], 0.0)
            t = jnp.maximum(jnp.dot(t, wr["rw2"][...]) + wr["rb2"][...], 0.0)
            et = jnp.dot(t, wr["rw3"][...]) + wr["rb3"][...]
            outs.append(et)
            agg = et if agg is None else agg + et
        inp2 = jnp.concatenate([hi, agg], axis=1)
        t = jnp.maximum(jnp.dot(inp2, wr["ow1"][...]) + wr["ob1"][...], 0.0)
        t = jnp.maximum(jnp.dot(t, wr["ow2"][...]) + wr["ob2"][...], 0.0)
        dh = jnp.dot(t, wr["ow3"][...]) + wr["ob3"][...]
        hn = alpha * hi + (1.0 - alpha) * dh
        hn16 = jnp.concatenate(
            [hn, jnp.zeros((tile, _HP - hd), jnp.float32)], axis=1)
        return hn16, jnp.concatenate(outs, axis=1)

    names = ["rw1", "rb1", "rw2", "rb2", "rw3", "rb3",
             "ow1", "ob1", "ow2", "ob2", "ow3", "ob3"]
    weights = [rw1, rb1[None, :], rw2, rb2[None, :], rw3, rb3[None, :],
               ow1, ob1[None, :], ow2, ob2[None, :], ow3, ob3[None, :]]
    if eenc is not None:
        (ew1, eb1), (ew2, eb2) = eenc
        names += ["ew1", "eb1", "ew2", "eb2"]
        weights += [ew1, eb1[None, :], ew2, eb2[None, :]]

    def body(*refs):
        n_out = 2 if want_e else 1
        if eenc is None:
            h_ref, hj_ref, e_ref = refs[:3]
            wrefs = refs[3:-n_out]
        else:
            h_ref, hj_ref = refs[:2]
            e_ref = None
            wrefs = refs[2:-n_out]
        wr = dict(zip(names, wrefs))
        hjs = [hj_ref[:, kk * _HP:kk * _HP + hd] for kk in range(k)]
        eks = (None if e_ref is None else
               [e_ref[:, kk * ed:(kk + 1) * ed] for kk in range(k)])
        hn16, e_new = compute(h_ref[...], hjs, eks, wr)
        if want_e:
            refs[-2][...] = hn16
            refs[-1][...] = e_new
        else:
            refs[-1][...] = hn16

    row = lambda w: pl.BlockSpec((tile, w), lambda i: (i, 0))
    full = lambda w: pl.BlockSpec(w.shape, lambda i: (0, 0))
    in_specs = [row(_HP), row(k * _HP)]
    operands = [h, hj]
    if eenc is None:
        in_specs.append(row(k * ed))
        operands.append(e)
    in_specs += [full(w) for w in weights]
    operands += weights
    out_specs = [row(_HP)]
    out_shape = [jax.ShapeDtypeStruct((NP, _HP), jnp.float32)]
    if want_e:
        out_specs.append(row(k * ed))
        out_shape.append(jax.ShapeDtypeStruct((NP, k * ed), jnp.float32))
    res = pl.pallas_call(
        body,
        grid=(NP // tile,),
        in_specs=in_specs,
        out_specs=out_specs,
        out_shape=out_shape,
    )(*operands)
    return res if want_e else (res[0], None)


def _final(h, bparams, xparams):
    NP = h.shape[0]
    (bw1, bb1), (bw2, bb2), (bw3, bb3) = bparams
    (xw1, xb1), (xw2, xb2), (xw3, xb3) = xparams
    hd = bw1.shape[0]
    tile = _TILE

    def body(h_ref,
             bw1r, bb1r, bw2r, bb2r, bw3r, bb3r,
             xw1r, xb1r, xw2r, xb2r, xw3r, xb3r,
             hout_ref, beta_ref):
        hi = h_ref[:, :hd]
        t = jnp.maximum(jnp.dot(hi, bw1r[...]) + bb1r[...], 0.0)
        t = jnp.maximum(jnp.dot(t, bw2r[...]) + bb2r[...], 0.0)
        blog = jnp.dot(t, bw3r[...]) + bb3r[...]
        beta_ref[...] = jax.nn.sigmoid(blog) + 1e-11
        t = jnp.maximum(jnp.dot(hi, xw1r[...]) + xb1r[...], 0.0)
        t = jnp.maximum(jnp.dot(t, xw2r[...]) + xb2r[...], 0.0)
        hout_ref[...] = jnp.dot(t, xw3r[...]) + xb3r[...]

    full = lambda w: pl.BlockSpec(w.shape, lambda i: (0, 0))
    bias = lambda b: pl.BlockSpec((1, b.shape[0]), lambda i: (0, 0))
    return pl.pallas_call(
        body,
        grid=(NP // tile,),
        in_specs=[
            pl.BlockSpec((tile, _HP), lambda i: (i, 0)),
            full(bw1), bias(bb1), full(bw2), bias(bb2), full(bw3), bias(bb3),
            full(xw1), bias(xb1), full(xw2), bias(xb2), full(xw3), bias(xb3),
        ],
        out_specs=[
            pl.BlockSpec((tile, xw3.shape[1]), lambda i: (i, 0)),
            pl.BlockSpec((tile, 1), lambda i: (i, 0)),
        ],
        out_shape=[
            jax.ShapeDtypeStruct((NP, xw3.shape[1]), jnp.float32),
            jax.ShapeDtypeStruct((NP, 1), jnp.float32),
        ],
    )(h,
      bw1, bb1[None, :], bw2, bb2[None, :], bw3, bb3[None, :],
      xw1, xb1[None, :], xw2, xb2[None, :], xw3, xb3[None, :])


def _block(bp, h_p, n, k, alpha):
    NP, dp = h_p.shape
    d = bp["node_encoder"][0][0].shape[0] // 2
    hr = h_p[:n, :d]
    sq = jnp.sum(hr * hr, axis=1)
    sq_p = jnp.pad(sq, (0, NP - n), constant_values=jnp.inf)

    nbr = _knn(h_p, sq_p, k)
    idx = nbr.reshape(-1)

    xj = _sc_gather(h_p, idx).reshape(NP, k * dp)
    h1 = _encoder(h_p, xj, bp["node_encoder"], k, dp)

    nl = len(bp["layers"])
    hj = _sc_gather(h1, idx).reshape(NP, k * _HP)
    hcur, e = _in_layer(h1, hj, None, bp["layers"][0], k, alpha,
                        eenc=bp["edge_encoder"], want_e=nl > 1)
    for li, lp in enumerate(bp["layers"][1:], start=1):
        hj = _sc_gather(hcur, idx).reshape(NP, k * _HP)
        hcur, e = _in_layer(hcur, hj, e, lp, k, alpha, want_e=li < nl - 1)
    return hcur


def kernel(x, params):
    alpha = 0.5
    nb = len(params["blocks"])
    ks = [nb - 1] + [nb - 1 - i for i in range(nb - 1)]
    n = x.shape[0]
    NP = _round_up(n, 512)
    h_p = jnp.pad(x, ((0, NP - n), (0, 0)))
    for bp, k in zip(params["blocks"], ks):
        h_p = _block(bp, h_p, n, k, alpha)
    h_out, beta = _final(h_p, params["B"], params["X"])
    return (h_out[:n], beta[:n])


# speedup vs baseline: 13.3222x; 1.0975x over previous
import functools

import jax
import jax.numpy as jnp
from jax import lax
from jax.experimental import pallas as pl
from jax.experimental.pallas import tpu as pltpu
from jax.experimental.pallas import tpu_sc as plsc

_TILE = 1024
_KTILE = 512
_HP = 16


def _round_up(a, m):
    return (a + m - 1) // m * m


def _knn(h_p, sq_p, k):
    NP, d = h_p.shape
    tile = _KTILE
    grid = NP // tile

    def body(h_blk, hT_ref, sqr, sqc, out_ref):
        pid = pl.program_id(0)
        p = jnp.dot(h_blk[...], hT_ref[...])
        s = (sqc[...] - 2.0 * p) + sqr[...]
        col = lax.broadcasted_iota(jnp.int32, (tile, NP), 1)
        row = lax.broadcasted_iota(jnp.int32, (tile, NP), 0) + pid * tile
        s = jnp.where(col == row, jnp.inf, s)
        cols_out = []
        for kk in range(k):
            idx = jnp.argmin(s, axis=1).astype(jnp.int32)
            cols_out.append(idx[:, None])
            if kk + 1 < k:
                s = jnp.where(col == idx[:, None], jnp.inf, s)
        out_ref[...] = jnp.concatenate(cols_out, axis=1)

    return pl.pallas_call(
        body,
        grid=(grid,),
        in_specs=[
            pl.BlockSpec((tile, d), lambda i: (i, 0)),
            pl.BlockSpec((d, NP), lambda i: (0, 0)),
            pl.BlockSpec((1, NP), lambda i: (0, 0)),
            pl.BlockSpec((tile, 1), lambda i: (i, 0)),
        ],
        out_specs=pl.BlockSpec((tile, k), lambda i: (i, 0)),
        out_shape=jax.ShapeDtypeStruct((NP, k), jnp.int32),
    )(h_p, h_p.T, sq_p[None, :], sq_p[:, None])


def _sc_gather(table, idx):
    B = idx.shape[0]
    V, D = table.shape
    info = plsc.get_sparse_core_info()
    nw = info.num_cores * info.num_subcores
    b_per_w = B // nw
    mesh = plsc.VectorSubcoreMesh(core_axis_name="c", subcore_axis_name="s")

    @functools.partial(
        pl.kernel,
        mesh=mesh,
        compiler_params=pltpu.CompilerParams(use_tc_tiling_on_sc=False),
        out_type=jax.ShapeDtypeStruct((B, D), jnp.float32),
        scratch_types=[
            pltpu.VMEM((b_per_w,), jnp.int32),
            pltpu.VMEM((b_per_w, D), jnp.float32),
            pltpu.SemaphoreType.DMA,
        ],
    )
    def gk(table_hbm, idx_hbm, out_hbm, idx_v, rows_v, sem):
        wid = lax.axis_index("s") * info.num_cores + lax.axis_index("c")
        base = wid * b_per_w
        pltpu.sync_copy(idx_hbm.at[pl.ds(base, b_per_w)], idx_v)
        pltpu.async_copy(table_hbm.at[idx_v], rows_v, sem).wait()
        pltpu.sync_copy(rows_v, out_hbm.at[pl.ds(base, b_per_w)])

    return gk(table, idx)


def _encoder(x_p, xj_cat, enc, k, dp):
    NP = x_p.shape[0]
    (w1, b1), (w2, b2) = enc
    d = w1.shape[0] // 2
    hd = w2.shape[1]
    tile = _TILE

    def body(xi_ref, xj_ref, w1r, b1r, w2r, b2r, out_ref):
        xi = xi_ref[:, :d]
        acc = None
        for kk in range(k):
            xjk = xj_ref[:, kk * dp:kk * dp + d]
            inp = jnp.concatenate([xi, xjk - xi], axis=1)
            hid = jnp.maximum(jnp.dot(inp, w1r[...]) + b1r[...], 0.0)
            msg = jnp.dot(hid, w2r[...]) + b2r[...]
            acc = msg if acc is None else acc + msg
        h = jnp.maximum(acc, 0.0)
        out_ref[...] = jnp.concatenate(
            [h, jnp.zeros((tile, _HP - hd), jnp.float32)], axis=1)

    return pl.pallas_call(
        body,
        grid=(NP // tile,),
        in_specs=[
            pl.BlockSpec((tile, x_p.shape[1]), lambda i: (i, 0)),
            pl.BlockSpec((tile, k * dp), lambda i: (i, 0)),
            pl.BlockSpec(w1.shape, lambda i: (0, 0)),
            pl.BlockSpec((1, b1.shape[0]), lambda i: (0, 0)),
            pl.BlockSpec(w2.shape, lambda i: (0, 0)),
            pl.BlockSpec((1, b2.shape[0]), lambda i: (0, 0)),
        ],
        out_specs=pl.BlockSpec((tile, _HP), lambda i: (i, 0)),
        out_shape=jax.ShapeDtypeStruct((NP, _HP), jnp.float32),
    )(x_p, xj_cat, w1, b1[None, :], w2, b2[None, :])


def _in_layer(h, hj, e, lp, k, alpha, eenc=None, want_e=True):
    NP = h.shape[0]
    (rw1, rb1), (rw2, rb2), (rw3, rb3) = lp["rel"]
    (ow1, ob1), (ow2, ob2), (ow3, ob3) = lp["obj"]
    ed = rw3.shape[1]
    hd = ow3.shape[1]
    tile = _TILE

    def compute(hi16, hjs, eks, wr):
        hi = hi16[:, :hd]
        outs = []
        agg = None
        for kk in range(k):
            hjk = hjs[kk]
            if eenc is None:
                ek = eks[kk]
            else:
                ei = jnp.concatenate([hjk, hi], axis=1)
                t = jnp.maximum(jnp.dot(ei, wr["ew1"][...]) + wr["eb1"][...], 0.0)
                ek = jnp.maximum(jnp.dot(t, wr["ew2"][...]) + wr["eb2"][...], 0.0)
            inp = jnp.concatenate([hi, hjk, ek], axis=1)
            t = jnp.maximum(jnp.dot(inp, wr["rw1"][...]) + wr["rb1"][...], 0.0)
            t = jnp.maximum(jnp.dot(t, wr["rw2"][...]) + wr["rb2"][...], 0.0)
            et = jnp.dot(t, wr["rw3"][...]) + wr["rb3"][...]
            outs.append(et)
            agg = et if agg is None else agg + et
        inp2 = jnp.concatenate([hi, agg], axis=1)
        t = jnp.maximum(jnp.dot(inp2, wr["ow1"][...]) + wr["ob1"][...], 0.0)
        t = jnp.maximum(jnp.dot(t, wr["ow2"][...]) + wr["ob2"][...], 0.0)
        dh = jnp.dot(t, wr["ow3"][...]) + wr["ob3"][...]
        hn = alpha * hi + (1.0 - alpha) * dh
        hn16 = jnp.concatenate(
            [hn, jnp.zeros((tile, _HP - hd), jnp.float32)], axis=1)
        return hn16, jnp.concatenate(outs, axis=1)

    names = ["rw1", "rb1", "rw2", "rb2", "rw3", "rb3",
             "ow1", "ob1", "ow2", "ob2", "ow3", "ob3"]
    weights = [rw1, rb1[None, :], rw2, rb2[None, :], rw3, rb3[None, :],
               ow1, ob1[None, :], ow2, ob2[None, :], ow3, ob3[None, :]]
    if eenc is not None:
        (ew1, eb1), (ew2, eb2) = eenc
        names += ["ew1", "eb1", "ew2", "eb2"]
        weights += [ew1, eb1[None, :], ew2, eb2[None, :]]

    def body(*refs):
        n_out = 2 if want_e else 1
        if eenc is None:
            h_ref, hj_ref, e_ref = refs[:3]
            wrefs = refs[3:-n_out]
        else:
            h_ref, hj_ref = refs[:2]
            e_ref = None
            wrefs = refs[2:-n_out]
        wr = dict(zip(names, wrefs))
        hjs = [hj_ref[:, kk * _HP:kk * _HP + hd] for kk in range(k)]
        eks = (None if e_ref is None else
               [e_ref[:, kk * ed:(kk + 1) * ed] for kk in range(k)])
        hn16, e_new = compute(h_ref[...], hjs, eks, wr)
        if want_e:
            refs[-2][...] = hn16
            refs[-1][...] = e_new
        else:
            refs[-1][...] = hn16

    row = lambda w: pl.BlockSpec((tile, w), lambda i: (i, 0))
    full = lambda w: pl.BlockSpec(w.shape, lambda i: (0, 0))
    in_specs = [row(_HP), row(k * _HP)]
    operands = [h, hj]
    if eenc is None:
        in_specs.append(row(k * ed))
        operands.append(e)
    in_specs += [full(w) for w in weights]
    operands += weights
    out_specs = [row(_HP)]
    out_shape = [jax.ShapeDtypeStruct((NP, _HP), jnp.float32)]
    if want_e:
        out_specs.append(row(k * ed))
        out_shape.append(jax.ShapeDtypeStruct((NP, k * ed), jnp.float32))
    res = pl.pallas_call(
        body,
        grid=(NP // tile,),
        in_specs=in_specs,
        out_specs=out_specs,
        out_shape=out_shape,
    )(*operands)
    return res if want_e else (res[0], None)


def _final(h, bparams, xparams):
    NP = h.shape[0]
    (bw1, bb1), (bw2, bb2), (bw3, bb3) = bparams
    (xw1, xb1), (xw2, xb2), (xw3, xb3) = xparams
    hd = bw1.shape[0]
    tile = _TILE

    def body(h_ref,
             bw1r, bb1r, bw2r, bb2r, bw3r, bb3r,
             xw1r, xb1r, xw2r, xb2r, xw3r, xb3r,
             hout_ref, beta_ref):
        hi = h_ref[:, :hd]
        t = jnp.maximum(jnp.dot(hi, bw1r[...]) + bb1r[...], 0.0)
        t = jnp.maximum(jnp.dot(t, bw2r[...]) + bb2r[...], 0.0)
        blog = jnp.dot(t, bw3r[...]) + bb3r[...]
        beta_ref[...] = jax.nn.sigmoid(blog) + 1e-11
        t = jnp.maximum(jnp.dot(hi, xw1r[...]) + xb1r[...], 0.0)
        t = jnp.maximum(jnp.dot(t, xw2r[...]) + xb2r[...], 0.0)
        hout_ref[...] = jnp.dot(t, xw3r[...]) + xb3r[...]

    full = lambda w: pl.BlockSpec(w.shape, lambda i: (0, 0))
    bias = lambda b: pl.BlockSpec((1, b.shape[0]), lambda i: (0, 0))
    return pl.pallas_call(
        body,
        grid=(NP // tile,),
        in_specs=[
            pl.BlockSpec((tile, _HP), lambda i: (i, 0)),
            full(bw1), bias(bb1), full(bw2), bias(bb2), full(bw3), bias(bb3),
            full(xw1), bias(xb1), full(xw2), bias(xb2), full(xw3), bias(xb3),
        ],
        out_specs=[
            pl.BlockSpec((tile, xw3.shape[1]), lambda i: (i, 0)),
            pl.BlockSpec((tile, 1), lambda i: (i, 0)),
        ],
        out_shape=[
            jax.ShapeDtypeStruct((NP, xw3.shape[1]), jnp.float32),
            jax.ShapeDtypeStruct((NP, 1), jnp.float32),
        ],
    )(h,
      bw1, bb1[None, :], bw2, bb2[None, :], bw3, bb3[None, :],
      xw1, xb1[None, :], xw2, xb2[None, :], xw3, xb3[None, :])


def _block(bp, h_p, n, k, alpha):
    NP, dp = h_p.shape
    d = bp["node_encoder"][0][0].shape[0] // 2
    hr = h_p[:n, :d]
    sq = jnp.sum(hr * hr, axis=1)
    sq_p = jnp.pad(sq, (0, NP - n), constant_values=jnp.inf)

    nbr = _knn(h_p, sq_p, k)
    idx = nbr.reshape(-1)

    xj = _sc_gather(h_p, idx).reshape(NP, k * dp)
    h1 = _encoder(h_p, xj, bp["node_encoder"], k, dp)

    nl = len(bp["layers"])
    hj = _sc_gather(h1, idx).reshape(NP, k * _HP)
    hcur, e = _in_layer(h1, hj, None, bp["layers"][0], k, alpha,
                        eenc=bp["edge_encoder"], want_e=nl > 1)
    for li, lp in enumerate(bp["layers"][1:], start=1):
        hj = _sc_gather(hcur, idx).reshape(NP, k * _HP)
        hcur, e = _in_layer(hcur, hj, e, lp, k, alpha, want_e=li < nl - 1)
    return hcur


def kernel(x, params):
    alpha = 0.5
    nb = len(params["blocks"])
    ks = [nb - 1] + [nb - 1 - i for i in range(nb - 1)]
    n = x.shape[0]
    NP = _round_up(n, 512)
    h_p = jnp.pad(x, ((0, NP - n), (0, 0)))
    for bp, k in zip(params["blocks"], ks):
        h_p = _block(bp, h_p, n, k, alpha)
    h_out, beta = _final(h_p, params["B"], params["X"])
    return (h_out[:n], beta[:n])
